# baseline (speedup 1.0000x reference)
import functools

import jax
import jax.numpy as jnp
from jax import lax
from jax.experimental import pallas as pl
from jax.experimental.pallas import tpu as pltpu
from jax.experimental.pallas import tpu_sc as plsc

VOCAB = 100000
EMBED = 128
B = 4
S = 2048
N = B * S
LN_EPS = 1e-5

NUM_CORES = 2
NUM_SUBCORES = 16
NW = NUM_CORES * NUM_SUBCORES
ROWS_PER_W = N // NW


S_PER_W = S // NW
L = 16


def _sc_gather_add(token_table, position_table, ids_flat):
    mesh = plsc.VectorSubcoreMesh(core_axis_name="c", subcore_axis_name="s")

    @functools.partial(
        pl.kernel,
        mesh=mesh,
        out_type=jax.ShapeDtypeStruct((N, EMBED), jnp.float32),
        scratch_types=[
            pltpu.VMEM((ROWS_PER_W,), jnp.int32),
            pltpu.VMEM((ROWS_PER_W, EMBED), jnp.float32),
            pltpu.VMEM((S_PER_W + 8, EMBED), jnp.float32),
            pltpu.SemaphoreType.DMA,
            pltpu.SemaphoreType.DMA,
        ],
    )
    def k(ids_hbm, table_hbm, pos_hbm, out_hbm, idx_v, rows_v, pos_v, sem, sem2):
        wid = lax.axis_index("s") * NUM_CORES + lax.axis_index("c")
        base = wid * ROWS_PER_W
        s0 = wid * S_PER_W
        pltpu.sync_copy(ids_hbm.at[pl.ds(base, ROWS_PER_W)], idx_v)
        cp_pos = pltpu.async_copy(pos_hbm.at[pl.ds(s0, S_PER_W + 8)], pos_v, sem2)
        pltpu.async_copy(table_hbm.at[idx_v], rows_v, sem).wait()
        cp_pos.wait()

        @pl.loop(0, ROWS_PER_W)
        def _(r):
            pr = lax.shift_right_logical(r, 2) + 2
            for j in range(EMBED // L):
                sl = pl.ds(j * L, L)
                rows_v[r, sl] = rows_v[r, sl] + pos_v[pr, sl]

        pltpu.sync_copy(rows_v, out_hbm.at[pl.ds(base, ROWS_PER_W)])

    return k(ids_flat, token_table, position_table)


S_BLK = 256


def _tc_ln_body(x_ref, g_ref, b_ref, o_ref):
    e = x_ref[...]
    mean = jnp.mean(e, axis=-1, keepdims=True)
    c = e - mean
    var = jnp.mean(c * c, axis=-1, keepdims=True)
    o_ref[...] = c * lax.rsqrt(var + LN_EPS) * g_ref[...] + b_ref[...]


def _tc_ln(summed, ln_gamma, ln_beta):
    return pl.pallas_call(
        _tc_ln_body,
        grid=(S // S_BLK,),
        in_specs=[
            pl.BlockSpec((S_BLK, B, EMBED), lambda i: (i, 0, 0)),
            pl.BlockSpec((EMBED,), lambda i: (0,)),
            pl.BlockSpec((EMBED,), lambda i: (0,)),
        ],
        out_specs=pl.BlockSpec((S_BLK, B, EMBED), lambda i: (i, 0, 0)),
        out_shape=jax.ShapeDtypeStruct((S, B, EMBED), jnp.float32),
    )(summed, ln_gamma, ln_beta)


def kernel(input_ids, token_table, position_table, ln_gamma, ln_beta):
    ids_flat = input_ids.astype(jnp.int32).T.reshape(-1)
    summed = _sc_gather_add(token_table, position_table, ids_flat)
    return _tc_ln(summed.reshape(S, B, EMBED), ln_gamma, ln_beta)


# speedup vs baseline: 1.2542x; 1.2542x over previous
import functools

import jax
import jax.numpy as jnp
from jax import lax
from jax.experimental import pallas as pl
from jax.experimental.pallas import tpu as pltpu
from jax.experimental.pallas import tpu_sc as plsc

VOCAB = 100000
EMBED = 128
B = 4
S = 2048
N = B * S
LN_EPS = 1e-5

NUM_CORES = 1
NUM_SUBCORES = 16
NW = NUM_CORES * NUM_SUBCORES
ROWS_PER_W = N // NW


def _sc_gather(token_table, ids_flat):
    mesh = plsc.VectorSubcoreMesh(
        core_axis_name="c", subcore_axis_name="s", num_cores=NUM_CORES
    )

    @functools.partial(
        pl.kernel,
        mesh=mesh,
        out_type=jax.ShapeDtypeStruct((N, EMBED), jnp.float32),
        scratch_types=[
            pltpu.VMEM((ROWS_PER_W,), jnp.int32),
            pltpu.VMEM((ROWS_PER_W, EMBED), jnp.float32),
            pltpu.SemaphoreType.DMA,
        ],
    )
    def k(ids_hbm, table_hbm, out_hbm, idx_v, rows_v, sem):
        wid = lax.axis_index("s") * NUM_CORES + lax.axis_index("c")
        base = wid * ROWS_PER_W
        pltpu.sync_copy(ids_hbm.at[pl.ds(base, ROWS_PER_W)], idx_v)
        pltpu.async_copy(table_hbm.at[idx_v], rows_v, sem).wait()
        pltpu.sync_copy(rows_v, out_hbm.at[pl.ds(base, ROWS_PER_W)])

    return k(ids_flat, token_table)


S_BLK = 256


def _tc_ln_body(x_ref, pos_ref, g_ref, b_ref, o_ref):
    x = x_ref[...]
    p = pos_ref[...]
    e = x + p[:, None, :]
    mean = jnp.mean(e, axis=-1, keepdims=True)
    c = e - mean
    var = jnp.mean(c * c, axis=-1, keepdims=True)
    o_ref[...] = c * lax.rsqrt(var + LN_EPS) * g_ref[...] + b_ref[...]


def _tc_ln(gathered, pos, ln_gamma, ln_beta):
    return pl.pallas_call(
        _tc_ln_body,
        grid=(S // S_BLK,),
        in_specs=[
            pl.BlockSpec((S_BLK, B, EMBED), lambda i: (i, 0, 0)),
            pl.BlockSpec((S_BLK, EMBED), lambda i: (i, 0)),
            pl.BlockSpec((EMBED,), lambda i: (0,)),
            pl.BlockSpec((EMBED,), lambda i: (0,)),
        ],
        out_specs=pl.BlockSpec((S_BLK, B, EMBED), lambda i: (i, 0, 0)),
        out_shape=jax.ShapeDtypeStruct((S, B, EMBED), jnp.float32),
    )(gathered, pos, ln_gamma, ln_beta)


def kernel(input_ids, token_table, position_table, ln_gamma, ln_beta):
    ids_flat = input_ids.astype(jnp.int32).T.reshape(-1)
    gathered = _sc_gather(token_table, ids_flat)
    pos = lax.slice(position_table, (2, 0), (2 + S, EMBED))
    return _tc_ln(gathered.reshape(S, B, EMBED), pos, ln_gamma, ln_beta)
